# baseline (speedup 1.0000x reference)
import functools

import jax
import jax.numpy as jnp
from jax import lax
from jax.experimental import pallas as pl
from jax.experimental.pallas import tpu as pltpu
from jax.experimental.pallas import tpu_sc as plsc

NUM_CORES = 2
NUM_SUBCORES = 16
NUM_WORKERS = NUM_CORES * NUM_SUBCORES
BLK = 128


@functools.lru_cache(maxsize=None)
def _make_kernel(NI: int, NJ: int, D: int):
    blocks_per_j = NI // BLK
    n_blocks = blocks_per_j * NJ
    per_w = n_blocks // NUM_WORKERS
    mesh = plsc.VectorSubcoreMesh(core_axis_name="c", subcore_axis_name="s")

    scratch = (
        [pltpu.VMEM((BLK,), jnp.int32) for _ in range(2)]
        + [pltpu.VMEM((BLK, D), jnp.float32) for _ in range(2)]
        + [pltpu.VMEM((D // 8, 1, 8, BLK), jnp.float32) for _ in range(2)]
        + [pltpu.SemaphoreType.DMA] * 6
    )

    @functools.partial(
        pl.kernel,
        mesh=mesh,
        out_type=jax.ShapeDtypeStruct((NJ, D // 8, blocks_per_j, 8, BLK),
                                      jnp.float32),
        scratch_types=scratch,
        compiler_params=pltpu.CompilerParams(
            use_tc_tiling_on_sc=False, needs_layout_passes=False),
    )
    def emb_kernel(xT_hbm, table_hbm, out_hbm, i0, i1, r0, r1, t0, t1,
                   si0, si1, sg0, sg1, ss0, ss1):
        idx_b = (i0, i1)
        rows_b = (r0, r1)
        rowsT_b = (t0, t1)
        si = (si0, si1)
        sg = (sg0, sg1)
        ss = (ss0, ss1)

        w = lax.axis_index("s") * NUM_CORES + lax.axis_index("c")
        base = w * per_w

        dim_sel = (lax.iota(jnp.int32, 16), lax.iota(jnp.int32, 16) + 16)
        a_sel = tuple(s // 8 for s in dim_sel)
        dd_sel = tuple(s % 8 for s in dim_sel)
        z_sel = jnp.zeros((16,), jnp.int32)

        def idx_cp(g, p):
            j = g // blocks_per_j
            b = g % blocks_per_j
            return pltpu.make_async_copy(
                xT_hbm.at[j, pl.ds(b * BLK, BLK)], idx_b[p], si[p])

        def gather_cp(p):
            return pltpu.make_async_copy(table_hbm.at[idx_b[p]], rows_b[p], sg[p])

        def store_cp(g, p):
            j = g // blocks_per_j
            b = g % blocks_per_j
            return pltpu.make_async_copy(
                rowsT_b[p], out_hbm.at[j, :, pl.ds(b, 1), :, :], ss[p])

        idx_cp(base + 0, 0).start()
        idx_cp(base + 1, 1).start()
        idx_cp(base + 0, 0).wait()
        gather_cp(0).start()

        def body(i, carry):
            for p in (0, 1):
                blk = 2 * i + p
                g = base + blk
                p1 = p ^ 1

                @pl.when(blk + 1 <= per_w - 1)
                def _():
                    idx_cp(g + 1, p1).wait()
                    gather_cp(p1).start()

                gather_cp(p).wait()

                @pl.when(blk + 2 <= per_w - 1)
                def _():
                    idx_cp(g + 2, p).start()

                @pl.when(blk >= 2)
                def _():
                    store_cp(g - 2, p).wait()

                @plsc.parallel_loop(0, BLK, 1, unroll=16)
                def _(r):
                    col_r = jnp.full((16,), r, jnp.int32)
                    for h in (0, 1):
                        v = rows_b[p][r, pl.ds(16 * h, 16)]
                        plsc.store_scatter(
                            rowsT_b[p], [a_sel[h], z_sel, dd_sel[h], col_r], v)

                store_cp(g, p).start()
            return carry

        lax.fori_loop(0, per_w // 2, body, 0)
        store_cp(base + per_w - 2, 0).wait()
        store_cp(base + per_w - 1, 1).wait()

    return emb_kernel


def kernel(x, embd):
    NI, NJ = x.shape
    D = embd.shape[1]
    xT = x.T.astype(jnp.int32)
    outP = _make_kernel(NI, NJ, D)(xT, embd)
    out = jnp.transpose(outP, (2, 4, 0, 1, 3)).reshape(NI, NJ, D)
    return out
